# baseline (speedup 1.0000x reference)
import functools
import jax
import jax.numpy as jnp
from jax import lax
from jax.experimental import pallas as pl
from jax.experimental.pallas import tpu as pltpu
from jax.experimental.pallas import tpu_sc as plsc

N = 10000
CH = 128
BLK = 1000
NP = 10112
RTD = 632
NA = 10112
RT = 632

NC = 2
NS = 16
NW = NC * NS
CK = 64
NCH = 160
TOTCH = 5120
F0 = 240
F1 = 80
HFP = 40
BSZ = 16
EP = NCH * CK
E_PAD = NW * EP

_mesh = plsc.VectorSubcoreMesh(
    core_axis_name="c", subcore_axis_name="s", num_cores=NC, num_subcores=NS)


def _stage_a_body(x_ref, w_ref, dp_ref, o_ref):
    deg = jnp.sum(dp_ref[...], axis=1)[:, None] + 1.0
    dinv = lax.rsqrt(deg)
    o_ref[...] = jnp.dot(x_ref[...], w_ref[...],
                         preferred_element_type=jnp.float32) * dinv


def _stage_b_body(p0_ref, p1_ref, xs_ref, w_ref, b_ref, dp_ref, o_ref):
    deg = jnp.sum(dp_ref[...], axis=1)[:, None] + 1.0
    dinv = lax.rsqrt(deg)
    h = jnp.maximum(dinv * (p0_ref[...] + p1_ref[...] + xs_ref[...]) + b_ref[...], 0.0)
    o_ref[...] = jnp.dot(h, w_ref[...], preferred_element_type=jnp.float32) * dinv


def _stage_c_body(p0_ref, p1_ref, xs_ref, b_ref, dp_ref, o_ref):
    deg = jnp.sum(dp_ref[...], axis=1)[:, None] + 1.0
    dinv = lax.rsqrt(deg)
    o_ref[...] = dinv * (p0_ref[...] + p1_ref[...] + xs_ref[...]) + b_ref[...]


def _row_spec():
    return pl.BlockSpec((BLK, CH), lambda i: (i, 0))


def _deg_spec():
    return pl.BlockSpec((BLK, NW), lambda i: (i, 0))


def _stage_a(x, W1, dp):
    return pl.pallas_call(
        _stage_a_body,
        grid=(N // BLK,),
        in_specs=[_row_spec(),
                  pl.BlockSpec((CH, CH), lambda i: (0, 0)),
                  _deg_spec()],
        out_specs=_row_spec(),
        out_shape=jax.ShapeDtypeStruct((N, CH), jnp.float32),
    )(x, W1, dp)


def _stage_b(p0, p1, xs, W2, b1, dp):
    return pl.pallas_call(
        _stage_b_body,
        grid=(N // BLK,),
        in_specs=[_row_spec(), _row_spec(), _row_spec(),
                  pl.BlockSpec((CH, CH), lambda i: (0, 0)),
                  pl.BlockSpec((1, CH), lambda i: (0, 0)),
                  _deg_spec()],
        out_specs=_row_spec(),
        out_shape=jax.ShapeDtypeStruct((N, CH), jnp.float32),
    )(p0, p1, xs, W2, b1, dp)


def _stage_c(p0, p1, xs, b2, dp):
    return pl.pallas_call(
        _stage_c_body,
        grid=(N // BLK,),
        in_specs=[_row_spec(), _row_spec(), _row_spec(),
                  pl.BlockSpec((1, CH), lambda i: (0, 0)),
                  _deg_spec()],
        out_specs=_row_spec(),
        out_shape=jax.ShapeDtypeStruct((N, CH), jnp.float32),
    )(p0, p1, xs, b2, dp)


def _unpack_sd(sdv, srcv, nrows):
    def row(r, _):
        for k in range(CK // 16):
            v = sdv[r, pl.ds(k * 16, 16)]
            if srcv is not None:
                srcv[r, pl.ds(k * 16, 16)] = v & 0xFFFF
            sdv[r, pl.ds(k * 16, 16)] = lax.shift_right_logical(v, 16)
        return 0
    lax.fori_loop(0, nrows, row, 0)


def _deg_body(sd2, ewf, out, sdv, ewv, degp):
    cid = lax.axis_index("c")
    sid = lax.axis_index("s")
    wid = cid * NS + sid

    pltpu.sync_copy(sd2.at[pl.ds(wid * NCH, NCH)], sdv)
    pltpu.sync_copy(ewf.at[pl.ds(wid * EP, EP)], ewv)
    _unpack_sd(sdv, None, NCH)

    def zz(r, _):
        degp[pl.ds(r * 16, 16)] = jnp.zeros((16,), jnp.float32)
        return 0
    lax.fori_loop(0, NP // 16, zz, 0)

    def row(r, _):
        for kk in range(CK // 16):
            d16 = sdv[r, pl.ds(kk * 16, 16)]
            w16 = ewv[pl.ds(r * CK + kk * 16, 16)]
            plsc.addupdate_scatter(degp, [d16], w16)
        return 0
    lax.fori_loop(0, NCH, row, 0)

    pltpu.sync_copy(degp, out.at[cid, sid])


@functools.partial(
    pl.kernel,
    out_type=jax.ShapeDtypeStruct((NC, NS, NP), jnp.float32),
    mesh=_mesh,
    scratch_types=[
        pltpu.VMEM((NCH, CK), jnp.int32),
        pltpu.VMEM((EP,), jnp.float32),
        pltpu.VMEM((NP,), jnp.float32),
    ],
    compiler_params=pltpu.CompilerParams(needs_layout_passes=False),
)
def _deg_kernel(sd2, ewf, out, sdv, ewv, degp):
    _deg_body(sd2, ewf, out, sdv, ewv, degp)


def _prop_body(xs, sd2, ewf, out, sdv, srcv, ewv, dstb, g0, g1, g2, g3, acc,
               gs0, gs1, gs2, gs3, ss0, ss1, ss2, ss3):
    cid = lax.axis_index("c")
    sid = lax.axis_index("s")
    bufs = (g0, g1, g2, g3)
    gsems = (gs0, gs1, gs2, gs3)
    ssems = (ss0, ss1, ss2, ss3)

    def zv(r, _):
        for g in bufs:
            for k in range(CH // 16):
                g[r, pl.ds(k * 16, 16)] = jnp.zeros((16,), jnp.float32)
        return 0
    lax.fori_loop(0, CK, zv, 0)
    r0 = sid * RT
    for i in range(RT // CK):
        pltpu.sync_copy(g0, acc.at[pl.ds(r0 + i * CK, CK)])
    pltpu.sync_copy(g0.at[pl.ds(0, RT % CK)],
                    acc.at[pl.ds(r0 + (RT // CK) * CK, RT % CK)])
    plsc.subcore_barrier()

    start0 = jnp.where(cid == 0, sid * F0, NS * F0 + sid * F1)
    npieces = jnp.where(cid == 0, F0 // HFP, F1 // HFP)
    nbat = HFP // 4

    def fire(c, k):
        pltpu.async_copy(xs.at[srcv.at[c]], bufs[k], gsems[k])

    def process(c, k):
        pltpu.make_async_copy(xs.at[srcv.at[0]], bufs[k], gsems[k]).wait()
        buf = bufs[k]

        def scale(i, _):
            for j in (4 * i, 4 * i + 1, 4 * i + 2, 4 * i + 3):
                ewb = plsc.load_gather(
                    ewv, [c * CK + j + jnp.zeros((16,), jnp.int32)])
                for kk in range(CH // 16):
                    buf[j, pl.ds(kk * 16, 16)] = buf[j, pl.ds(kk * 16, 16)] * ewb
            return 0
        lax.fori_loop(0, CK // 4, scale, 0)
        for g in range(CK // 16):
            dstb[k, pl.ds(g * 16, 16)] = lax.shift_right_logical(
                sdv[c, pl.ds(g * 16, 16)], 16)
        pltpu.async_copy(buf, acc.at[dstb.at[k]], ssems[k], add=True)

    def wait_scatter(k):
        pltpu.make_async_copy(bufs[k], acc.at[dstb.at[0]], ssems[k]).wait()

    def piece(h, _):
        c0 = start0 + h * HFP
        pltpu.sync_copy(sd2.at[pl.ds(c0, HFP)], sdv)
        pltpu.sync_copy(ewf.at[pl.ds(c0 * CK, HFP * CK)], ewv)

        def unp(r, _):
            for g in range(CK // 16):
                srcv[r, pl.ds(g * 16, 16)] = sdv[r, pl.ds(g * 16, 16)] & 0xFFFF
            return 0
        lax.fori_loop(0, HFP, unp, 0)

        for k in range(4):
            fire(k, k)

        def batch(t, _):
            for k in range(4):
                process(4 * t + k, k)
            for k in range(4):
                wait_scatter(k)
                fire(4 * t + 4 + k, k)
            return 0

        lax.fori_loop(0, nbat - 1, batch, 0)
        for k in range(4):
            process((nbat - 1) * 4 + k, k)
            wait_scatter(k)
        return 0

    lax.fori_loop(0, npieces, piece, 0)

    plsc.subcore_barrier()
    pltpu.sync_copy(acc.at[pl.ds(r0, RT)], out.at[cid, pl.ds(r0, RT)])


@functools.partial(
    pl.kernel,
    out_type=jax.ShapeDtypeStruct((NC, NA, CH), jnp.float32),
    mesh=_mesh,
    scratch_types=[
        pltpu.VMEM((HFP, CK), jnp.int32),
        pltpu.VMEM((HFP, CK), jnp.int32),
        pltpu.VMEM((HFP * CK,), jnp.float32),
        pltpu.VMEM((4, CK), jnp.int32),
        pltpu.VMEM((CK, CH), jnp.float32),
        pltpu.VMEM((CK, CH), jnp.float32),
        pltpu.VMEM((CK, CH), jnp.float32),
        pltpu.VMEM((CK, CH), jnp.float32),
        pltpu.VMEM_SHARED((NA, CH), jnp.float32),
        pltpu.SemaphoreType.DMA,
        pltpu.SemaphoreType.DMA,
        pltpu.SemaphoreType.DMA,
        pltpu.SemaphoreType.DMA,
        pltpu.SemaphoreType.DMA,
        pltpu.SemaphoreType.DMA,
        pltpu.SemaphoreType.DMA,
        pltpu.SemaphoreType.DMA,
    ],
    compiler_params=pltpu.CompilerParams(needs_layout_passes=False),
)
def _prop_kernel(xs, sd2, ewf, out, sdv, srcv, ewv, dstb, g0, g1, g2, g3, acc,
                 gs0, gs1, gs2, gs3, ss0, ss1, ss2, ss3):
    _prop_body(xs, sd2, ewf, out, sdv, srcv, ewv, dstb, g0, g1, g2, g3, acc,
               gs0, gs1, gs2, gs3, ss0, ss1, ss2, ss3)


def kernel(x, edge_index, edge_weight, W1, b1, W2, b2):
    src = edge_index[0].astype(jnp.int32)
    dst = edge_index[1].astype(jnp.int32)
    ew = edge_weight.astype(jnp.float32)
    npad = E_PAD + 128 * CK - src.shape[0]
    sd = jnp.concatenate([(dst << 16) | src, jnp.zeros((npad,), jnp.int32)])
    sd2 = sd.reshape(TOTCH + 128, CK)
    ewf = jnp.concatenate([ew, jnp.zeros((npad,), jnp.float32)])
    b1r = b1.reshape(1, CH)
    b2r = b2.reshape(1, CH)

    dp = _deg_kernel(sd2, ewf).reshape(NW, NP).T
    xs1 = _stage_a(x, W1, dp)
    p = _prop_kernel(xs1, sd2, ewf)
    xs2 = _stage_b(p[0], p[1], xs1, W2, b1r, dp)
    q = _prop_kernel(xs2, sd2, ewf)
    return _stage_c(q[0], q[1], xs2, b2r, dp)


# speedup vs baseline: 1.3541x; 1.3541x over previous
import functools
import jax
import jax.numpy as jnp
from jax import lax
from jax.experimental import pallas as pl
from jax.experimental.pallas import tpu as pltpu
from jax.experimental.pallas import tpu_sc as plsc

N = 10000
CH = 128
BLK = 1000
NP = 10112
RTD = 632
NA = 10112
RT = 632

NC = 2
NS = 16
NW = NC * NS
CK = 64
NCH = 160
TOTCH = 5120
F0 = 256
F1 = 64
HFP = 40
BSZ = 16
EP = NCH * CK
E_PAD = NW * EP

_mesh = plsc.VectorSubcoreMesh(
    core_axis_name="c", subcore_axis_name="s", num_cores=NC, num_subcores=NS)


def _stage_a_body(x_ref, w_ref, dp_ref, o_ref):
    deg = jnp.sum(dp_ref[...], axis=1)[:, None] + 1.0
    dinv = lax.rsqrt(deg)
    o_ref[...] = jnp.dot(x_ref[...], w_ref[...],
                         preferred_element_type=jnp.float32) * dinv


def _stage_b_body(p0_ref, p1_ref, xs_ref, w_ref, b_ref, dp_ref, o_ref):
    deg = jnp.sum(dp_ref[...], axis=1)[:, None] + 1.0
    dinv = lax.rsqrt(deg)
    h = jnp.maximum(dinv * (p0_ref[...] + p1_ref[...] + xs_ref[...]) + b_ref[...], 0.0)
    o_ref[...] = jnp.dot(h, w_ref[...], preferred_element_type=jnp.float32) * dinv


def _stage_c_body(p0_ref, p1_ref, xs_ref, b_ref, dp_ref, o_ref):
    deg = jnp.sum(dp_ref[...], axis=1)[:, None] + 1.0
    dinv = lax.rsqrt(deg)
    o_ref[...] = dinv * (p0_ref[...] + p1_ref[...] + xs_ref[...]) + b_ref[...]


def _row_spec():
    return pl.BlockSpec((BLK, CH), lambda i: (i, 0))


def _deg_spec():
    return pl.BlockSpec((BLK, NW), lambda i: (i, 0))


def _stage_a(x, W1, dp):
    return pl.pallas_call(
        _stage_a_body,
        grid=(N // BLK,),
        in_specs=[_row_spec(),
                  pl.BlockSpec((CH, CH), lambda i: (0, 0)),
                  _deg_spec()],
        out_specs=_row_spec(),
        out_shape=jax.ShapeDtypeStruct((N, CH), jnp.float32),
    )(x, W1, dp)


def _stage_b(p0, p1, xs, W2, b1, dp):
    return pl.pallas_call(
        _stage_b_body,
        grid=(N // BLK,),
        in_specs=[_row_spec(), _row_spec(), _row_spec(),
                  pl.BlockSpec((CH, CH), lambda i: (0, 0)),
                  pl.BlockSpec((1, CH), lambda i: (0, 0)),
                  _deg_spec()],
        out_specs=_row_spec(),
        out_shape=jax.ShapeDtypeStruct((N, CH), jnp.float32),
    )(p0, p1, xs, W2, b1, dp)


def _stage_c(p0, p1, xs, b2, dp):
    return pl.pallas_call(
        _stage_c_body,
        grid=(N // BLK,),
        in_specs=[_row_spec(), _row_spec(), _row_spec(),
                  pl.BlockSpec((1, CH), lambda i: (0, 0)),
                  _deg_spec()],
        out_specs=_row_spec(),
        out_shape=jax.ShapeDtypeStruct((N, CH), jnp.float32),
    )(p0, p1, xs, b2, dp)


def _unpack_sd(sdv, srcv, nrows):
    def row(r, _):
        for k in range(CK // 16):
            v = sdv[r, pl.ds(k * 16, 16)]
            if srcv is not None:
                srcv[r, pl.ds(k * 16, 16)] = v & 0xFFFF
            sdv[r, pl.ds(k * 16, 16)] = lax.shift_right_logical(v, 16)
        return 0
    lax.fori_loop(0, nrows, row, 0)


def _deg_body(sd2, ewf, out, sdv, ewv, degp):
    cid = lax.axis_index("c")
    sid = lax.axis_index("s")
    wid = cid * NS + sid

    pltpu.sync_copy(sd2.at[pl.ds(wid * NCH, NCH)], sdv)
    pltpu.sync_copy(ewf.at[pl.ds(wid * EP, EP)], ewv)
    _unpack_sd(sdv, None, NCH)

    def zz(r, _):
        degp[pl.ds(r * 16, 16)] = jnp.zeros((16,), jnp.float32)
        return 0
    lax.fori_loop(0, NP // 16, zz, 0)

    def row(r, _):
        for kk in range(CK // 16):
            d16 = sdv[r, pl.ds(kk * 16, 16)]
            w16 = ewv[pl.ds(r * CK + kk * 16, 16)]
            plsc.addupdate_scatter(degp, [d16], w16)
        return 0
    lax.fori_loop(0, NCH, row, 0)

    pltpu.sync_copy(degp, out.at[cid, sid])


@functools.partial(
    pl.kernel,
    out_type=jax.ShapeDtypeStruct((NC, NS, NP), jnp.float32),
    mesh=_mesh,
    scratch_types=[
        pltpu.VMEM((NCH, CK), jnp.int32),
        pltpu.VMEM((EP,), jnp.float32),
        pltpu.VMEM((NP,), jnp.float32),
    ],
    compiler_params=pltpu.CompilerParams(needs_layout_passes=False),
)
def _deg_kernel(sd2, ewf, out, sdv, ewv, degp):
    _deg_body(sd2, ewf, out, sdv, ewv, degp)


def _prop_body(xs, sd2, ewf, out, sdv, srcv, ewv, dstb, g0, g1, g2, g3, acc,
               gs0, gs1, gs2, gs3, ss0, ss1, ss2, ss3):
    cid = lax.axis_index("c")
    sid = lax.axis_index("s")
    bufs = (g0, g1, g2, g3)
    gsems = (gs0, gs1, gs2, gs3)
    ssems = (ss0, ss1, ss2, ss3)

    def zv(r, _):
        for g in bufs:
            for k in range(CH // 16):
                g[r, pl.ds(k * 16, 16)] = jnp.zeros((16,), jnp.float32)
        return 0
    lax.fori_loop(0, CK, zv, 0)
    r0 = sid * RT
    for i in range(RT // CK):
        pltpu.sync_copy(g0, acc.at[pl.ds(r0 + i * CK, CK)])
    pltpu.sync_copy(g0.at[pl.ds(0, RT % CK)],
                    acc.at[pl.ds(r0 + (RT // CK) * CK, RT % CK)])
    plsc.subcore_barrier()

    start0 = jnp.where(cid == 0, sid * F0, NS * F0 + sid * F1)
    npieces = jnp.where(cid == 0, F0 // HFP, F1 // HFP)
    nbat = HFP // 4

    def fire(c, k):
        pltpu.async_copy(xs.at[srcv.at[c]], bufs[k], gsems[k])

    def process(c, k):
        pltpu.make_async_copy(xs.at[srcv.at[0]], bufs[k], gsems[k]).wait()
        buf = bufs[k]

        def scale(i, _):
            for j in (4 * i, 4 * i + 1, 4 * i + 2, 4 * i + 3):
                ewb = plsc.load_gather(
                    ewv, [c * CK + j + jnp.zeros((16,), jnp.int32)])
                for kk in range(CH // 16):
                    buf[j, pl.ds(kk * 16, 16)] = buf[j, pl.ds(kk * 16, 16)] * ewb
            return 0
        lax.fori_loop(0, CK // 4, scale, 0)
        for g in range(CK // 16):
            dstb[k, pl.ds(g * 16, 16)] = lax.shift_right_logical(
                sdv[c, pl.ds(g * 16, 16)], 16)
        pltpu.async_copy(buf, acc.at[dstb.at[k]], ssems[k], add=True)

    def wait_scatter(k):
        pltpu.make_async_copy(bufs[k], acc.at[dstb.at[0]], ssems[k]).wait()

    def piece(h, _):
        c0 = start0 + h * HFP
        pltpu.sync_copy(sd2.at[pl.ds(c0, HFP)], sdv)
        pltpu.sync_copy(ewf.at[pl.ds(c0 * CK, HFP * CK)], ewv)

        def unp(r, _):
            for g in range(CK // 16):
                srcv[r, pl.ds(g * 16, 16)] = sdv[r, pl.ds(g * 16, 16)] & 0xFFFF
            return 0
        lax.fori_loop(0, HFP, unp, 0)

        for k in range(4):
            fire(k, k)

        def batch(t, _):
            for k in range(4):
                process(4 * t + k, k)
            for k in range(4):
                wait_scatter(k)
                fire(4 * t + 4 + k, k)
            return 0

        lax.fori_loop(0, nbat - 1, batch, 0)
        for k in range(4):
            process((nbat - 1) * 4 + k, k)
            wait_scatter(k)
        return 0

    lax.fori_loop(0, npieces, piece, 0)

    plsc.subcore_barrier()
    pltpu.sync_copy(acc.at[pl.ds(r0, RT)], out.at[cid, pl.ds(r0, RT)])


@functools.partial(
    pl.kernel,
    out_type=jax.ShapeDtypeStruct((NC, NA, CH), jnp.float32),
    mesh=_mesh,
    scratch_types=[
        pltpu.VMEM((HFP, CK), jnp.int32),
        pltpu.VMEM((HFP, CK), jnp.int32),
        pltpu.VMEM((HFP * CK,), jnp.float32),
        pltpu.VMEM((4, CK), jnp.int32),
        pltpu.VMEM((CK, CH), jnp.float32),
        pltpu.VMEM((CK, CH), jnp.float32),
        pltpu.VMEM((CK, CH), jnp.float32),
        pltpu.VMEM((CK, CH), jnp.float32),
        pltpu.VMEM_SHARED((NA, CH), jnp.float32),
        pltpu.SemaphoreType.DMA,
        pltpu.SemaphoreType.DMA,
        pltpu.SemaphoreType.DMA,
        pltpu.SemaphoreType.DMA,
        pltpu.SemaphoreType.DMA,
        pltpu.SemaphoreType.DMA,
        pltpu.SemaphoreType.DMA,
        pltpu.SemaphoreType.DMA,
    ],
    compiler_params=pltpu.CompilerParams(needs_layout_passes=False),
)
def _prop_kernel(xs, sd2, ewf, out, sdv, srcv, ewv, dstb, g0, g1, g2, g3, acc,
                 gs0, gs1, gs2, gs3, ss0, ss1, ss2, ss3):
    _prop_body(xs, sd2, ewf, out, sdv, srcv, ewv, dstb, g0, g1, g2, g3, acc,
               gs0, gs1, gs2, gs3, ss0, ss1, ss2, ss3)


def kernel(x, edge_index, edge_weight, W1, b1, W2, b2):
    src = edge_index[0].astype(jnp.int32)
    dst = edge_index[1].astype(jnp.int32)
    ew = edge_weight.astype(jnp.float32)
    npad = E_PAD + 128 * CK - src.shape[0]
    sd = jnp.concatenate([(dst << 16) | src, jnp.zeros((npad,), jnp.int32)])
    sd2 = sd.reshape(TOTCH + 128, CK)
    ewf = jnp.concatenate([ew, jnp.zeros((npad,), jnp.float32)])
    b1r = b1.reshape(1, CH)
    b2r = b2.reshape(1, CH)

    dp = _deg_kernel(sd2, ewf).reshape(NW, NP).T
    xs1 = _stage_a(x, W1, dp)
    p = _prop_kernel(xs1, sd2, ewf)
    xs2 = _stage_b(p[0], p[1], xs1, W2, b1r, dp)
    q = _prop_kernel(xs2, sd2, ewf)
    return _stage_c(q[0], q[1], xs2, b2r, dp)
